# baseline (speedup 1.0000x reference)
import jax
import jax.numpy as jnp
from jax import lax
from jax.experimental import pallas as pl
from jax.experimental.pallas import tpu as pltpu
from jax.experimental.pallas import tpu_sc as plsc

NC, NS = 2, 16
NW = NC * NS
D0, D1 = 32, 16
D = D0 + D1
NBUF = 2


def _sc_body(j_hbm, t0_hbm, t1_hbm, out_hbm, idx_all, rows0, rows1,
             gsem0, gsem1, gsem2, gsem3, wsem0, wsem1, wsem2, wsem3):
    wid = lax.axis_index("s") * NC + lax.axis_index("c")
    pairs_total = j_hbm.shape[0]
    ppw = pairs_total // NW
    units = ppw * 8
    gsems = (gsem0, gsem1, gsem2, gsem3)
    wsems = (wsem0, wsem1, wsem2, wsem3)

    pltpu.sync_copy(j_hbm.at[pl.ds(wid * ppw, ppw)], idx_all)

    def unit_coords(u):
        p = u >> 3
        ls = u & 7
        pg = wid * ppw + p
        lt = pg >> 5
        bb = pg & 31
        return p, ls, lt * 8 + ls, bb

    def fire_gathers(u, s):
        p, ls, _, _ = unit_coords(u)
        src = idx_all.at[p, ls]
        pltpu.async_copy(t0_hbm.at[src], rows0.at[s], gsems[s])
        pltpu.async_copy(t1_hbm.at[src], rows1.at[s], gsems[s])

    def wait_gathers(s):
        pltpu.make_async_copy(t0_hbm.at[idx_all.at[0, 0]], rows0.at[s],
                              gsems[s]).wait()
        pltpu.make_async_copy(t1_hbm.at[idx_all.at[0, 0]], rows1.at[s],
                              gsems[s]).wait()

    def fire_writes(u, s):
        _, _, l, bb = unit_coords(u)
        pltpu.async_copy(rows0.at[s],
                         out_hbm.at[pl.ds(bb * 128, 128), l, pl.ds(0, D0)],
                         wsems[s])
        pltpu.async_copy(rows1.at[s],
                         out_hbm.at[pl.ds(bb * 128, 128), l, pl.ds(D0, D1)],
                         wsems[s])

    def drain_writes(s):
        pltpu.make_async_copy(rows0.at[s],
                              out_hbm.at[pl.ds(0, 128), 0, pl.ds(0, D0)],
                              wsems[s]).wait()
        pltpu.make_async_copy(rows1.at[s],
                              out_hbm.at[pl.ds(0, 128), 0, pl.ds(D0, D1)],
                              wsems[s]).wait()

    fire_gathers(0, 0)
    fire_gathers(1, 1)

    @pl.loop(0, units, step=4)
    def _quad(base):
        for k in range(4):
            u = base + k
            s = k
            s2 = (k + 2) % 4

            @pl.when(jnp.logical_and(u >= 2, u + 2 < units))
            def _():
                drain_writes(s2)

            @pl.when(u + 2 < units)
            def _():
                fire_gathers(u + 2, s2)

            wait_gathers(s)
            fire_writes(u, s)

    for s in range(4):
        drain_writes(s)


def kernel(inputs, table0, table1):
    B, L = inputs.shape
    J = (inputs.T.reshape(L // 8, 8, B // 128, 128)
         .transpose(0, 2, 1, 3)
         .reshape((L // 8) * (B // 128), 8, 128))
    mesh = plsc.VectorSubcoreMesh(core_axis_name="c", subcore_axis_name="s")
    out = pl.kernel(
        _sc_body,
        out_type=jax.ShapeDtypeStruct((B, L, D), jnp.float32),
        mesh=mesh,
        compiler_params=pltpu.CompilerParams(use_tc_tiling_on_sc=False),
        scratch_types=[
            pltpu.VMEM((25, 8, 128), jnp.int32),
            pltpu.VMEM((4, 128, D0), jnp.float32),
            pltpu.VMEM((4, 128, D1), jnp.float32),
        ] + [pltpu.SemaphoreType.DMA] * 8,
    )(J, table0, table1)
    return out
